# baseline (speedup 1.0000x reference)
import functools

import jax
import jax.numpy as jnp
from jax import lax
from jax.experimental import pallas as pl
from jax.experimental.pallas import tpu as pltpu
from jax.experimental.pallas import tpu_sc as plsc

NC = 2
NS = 16
NW = NC * NS
BPW = 128
NBUF = 4
S0, S1 = 104, 96


@functools.lru_cache(maxsize=None)
def _make_lookup(batch: int, seq: int, hidden: int):
    assert batch == NW * BPW and seq == S0 + S1
    seq_pad = 208
    mesh = plsc.VectorSubcoreMesh(core_axis_name="c", subcore_axis_name="s")

    @functools.partial(
        pl.kernel,
        mesh=mesh,
        out_type=jax.ShapeDtypeStruct((batch, seq, hidden), jnp.float32),
        scratch_types=[
            pltpu.VMEM((seq_pad, BPW), jnp.int32),
            pltpu.VMEM((BPW, seq_pad), jnp.int32),
            pltpu.VMEM((seq, hidden), jnp.float32),
            pltpu.VMEM((seq, hidden), jnp.float32),
            pltpu.VMEM((seq, hidden), jnp.float32),
            pltpu.VMEM((seq, hidden), jnp.float32),
            pltpu.SemaphoreType.DMA,
            pltpu.SemaphoreType.DMA,
            pltpu.SemaphoreType.DMA,
            pltpu.SemaphoreType.DMA,
        ],
        compiler_params=pltpu.CompilerParams(
            use_tc_tiling_on_sc=False, needs_layout_passes=False),
    )
    def lookup(xt_hbm, table_hbm, out_hbm, idx_v, idx_t, b0, b1, b2, b3,
               s0, s1, s2, s3):
        bufs = (b0, b1, b2, b3)
        sems = (s0, s1, s2, s3)
        wid = lax.axis_index("s") * NC + lax.axis_index("c")
        base = wid * BPW
        pltpu.sync_copy(xt_hbm.at[:, pl.ds(base, BPW)],
                        idx_v.at[pl.ds(0, seq)])

        lanes = jnp.arange(16, dtype=jnp.int32)
        row_vecs = [lanes + (16 * k) for k in range(seq_pad // 16)]

        def tbody(b, carry):
            col = jnp.zeros((16,), jnp.int32) + b
            for k in range(seq_pad // 16):
                vec = plsc.load_gather(idx_v, [row_vecs[k], col])
                idx_t[b, pl.ds(16 * k, 16)] = vec
            return carry

        lax.fori_loop(0, BPW, tbody, 0)

        def fire(b, buf, sem):
            pltpu.async_copy(table_hbm.at[idx_t.at[b, pl.ds(0, S0)]],
                             buf.at[pl.ds(0, S0)], sem)
            pltpu.async_copy(table_hbm.at[idx_t.at[b, pl.ds(S0, S1)]],
                             buf.at[pl.ds(S0, S1)], sem)

        def drain(buf, sem):
            pltpu.make_async_copy(table_hbm.at[pl.ds(0, S0)],
                                  buf.at[pl.ds(0, S0)], sem).wait()
            pltpu.make_async_copy(table_hbm.at[pl.ds(0, S1)],
                                  buf.at[pl.ds(S0, S1)], sem).wait()

        def write(b, buf):
            pltpu.sync_copy(buf, out_hbm.at[base + b])

        for i in range(NBUF):
            fire(i, bufs[i], sems[i])

        def body(t, carry):
            for i in range(NBUF):
                b = NBUF * t + i
                drain(bufs[i], sems[i])
                write(b, bufs[i])
                fire(b + NBUF, bufs[i], sems[i])
            return carry

        lax.fori_loop(0, BPW // NBUF - 1, body, 0)
        for i in range(NBUF):
            drain(bufs[i], sems[i])
            write(BPW - NBUF + i, bufs[i])

    return lookup


def kernel(x, table):
    batch, seq = x.shape
    hidden = table.shape[1]
    fn = _make_lookup(batch, seq, hidden)
    return fn(jnp.transpose(x), table)
